# baseline (speedup 1.0000x reference)
import functools

import jax
import jax.numpy as jnp
from jax import lax
from jax.experimental import pallas as pl
from jax.experimental.pallas import tpu as pltpu
from jax.experimental.pallas import tpu_sc as plsc
from jax.experimental import layout as _jl

NC, NS, LANES = 2, 16, 16
NW = NC * NS
B, S, E = 1024, 200, 64
SEQ_W = B // NW
IDX_SPLIT = ((0, 104), (104, 96))
SEQ_CH = 2
NCHUNK = SEQ_W // SEQ_CH
CH_BYTES = SEQ_CH * S * E * 4

_mesh = plsc.VectorSubcoreMesh(core_axis_name="c", subcore_axis_name="s")


@functools.partial(
    pl.kernel,
    out_type=jax.ShapeDtypeStruct((B, S, E), jnp.float32),
    mesh=_mesh,
    scratch_types=[
        pltpu.VMEM((SEQ_W, S), jnp.int32),
        pltpu.VMEM((SEQ_W, S), jnp.int32),
        pltpu.VMEM((SEQ_W, S), jnp.int32),
        pltpu.VMEM((2, SEQ_CH, S, E), jnp.float32),
        pltpu.SemaphoreType.DMA,
        pltpu.SemaphoreType.DMA,
        pltpu.SemaphoreType.DMA,
    ],
    compiler_params=pltpu.CompilerParams(use_tc_tiling_on_sc=False),
)
def _sc_embed(seq_hbm, seg_hbm, tok_hbm, comb_hbm, out_hbm,
              tok_idx_v, seg_raw_v, cmb_idx_v, rows_v, sem_g1, sem_g2, sem_out):
    cid = lax.axis_index("c")
    sid = lax.axis_index("s")
    wid = sid * NC + cid
    b0 = wid * SEQ_W

    pltpu.sync_copy(seq_hbm.at[pl.ds(b0, SEQ_W)], tok_idx_v)
    pltpu.sync_copy(seg_hbm.at[pl.ds(b0, SEQ_W)], seg_raw_v)

    lane = lax.iota(jnp.int32, LANES)
    col0 = [j * LANES for j in range(S // LANES)] + [S - LANES]

    def idx_body(r, _):
        for j0 in col0:
            pos3 = (lane + j0) * 3
            cmb_idx_v[r, pl.ds(j0, LANES)] = pos3 + seg_raw_v[r, pl.ds(j0, LANES)]
        return 0

    lax.fori_loop(0, SEQ_W, idx_body, 0)

    def fire_g1(c, p):
        s0 = c * SEQ_CH
        for s in range(SEQ_CH):
            for off, w in IDX_SPLIT:
                pltpu.async_copy(
                    comb_hbm.at[cmb_idx_v.at[s0 + s, pl.ds(off, w)]],
                    rows_v.at[p, s, pl.ds(off, w)], sem_g1)

    def drain(sem):
        pltpu.make_async_copy(out_hbm.at[pl.ds(0, SEQ_CH)], rows_v.at[0], sem).wait()

    fire_g1(0, 0)

    def chunk_body(c, _):
        p = lax.rem(c, 2)
        s0 = c * SEQ_CH
        drain(sem_g1)
        for s in range(SEQ_CH):
            for off, w in IDX_SPLIT:
                pltpu.async_copy(
                    tok_hbm.at[tok_idx_v.at[s0 + s, pl.ds(off, w)]],
                    rows_v.at[p, s, pl.ds(off, w)], sem_g2, add=True)

        @pl.when(c >= 1)
        def _():
            drain(sem_out)

        @pl.when(c <= NCHUNK - 2)
        def _():
            fire_g1(c + 1, 1 - p)

        drain(sem_g2)
        pltpu.async_copy(rows_v.at[p], out_hbm.at[pl.ds(b0 + s0, SEQ_CH)], sem_out)
        return 0

    lax.fori_loop(0, NCHUNK, chunk_body, 0)
    drain(sem_out)


@jax.jit
def kernel(sequence, segment_label, token_table, segment_table, pe):
    comb = (pe[:, None, :] + segment_table[None, :, :]).reshape(S * 3, E)
    out = _sc_embed(sequence.astype(jnp.int32), segment_label.astype(jnp.int32),
                    token_table, comb)
    return _jl.with_layout_constraint(
        out, _jl.Layout(major_to_minor=(0, 1, 2), tiling=((8,),)))


# speedup vs baseline: 1.3806x; 1.3806x over previous
import functools

import jax
import jax.numpy as jnp
from jax import lax
from jax.experimental import pallas as pl
from jax.experimental.pallas import tpu as pltpu
from jax.experimental.pallas import tpu_sc as plsc
from jax.experimental import layout as _jl

NC, NS, LANES = 2, 16, 16
NW = NC * NS
B, S, E = 1024, 200, 64
SEQ_W = B // NW
IDX_SPLIT = ((0, 104), (104, 96))
SEQ_CH = 2
NCHUNK = SEQ_W // SEQ_CH

_mesh = plsc.VectorSubcoreMesh(core_axis_name="c", subcore_axis_name="s")


@functools.partial(
    pl.kernel,
    out_type=jax.ShapeDtypeStruct((B, S, E), jnp.float32),
    mesh=_mesh,
    scratch_types=[
        pltpu.VMEM((SEQ_W, S), jnp.int32),
        pltpu.VMEM((SEQ_W, S), jnp.int32),
        pltpu.VMEM((SEQ_W, S), jnp.int32),
        pltpu.VMEM((2, SEQ_CH, S, E), jnp.float32),
        pltpu.SemaphoreType.DMA,
        pltpu.SemaphoreType.DMA,
        pltpu.SemaphoreType.DMA,
    ],
    compiler_params=pltpu.CompilerParams(use_tc_tiling_on_sc=False),
)
def _sc_embed(seq_hbm, seg_hbm, tok_hbm, comb_hbm, out_hbm,
              tok_idx_v, seg_raw_v, cmb_idx_v, rows_v, sem_g1, sem_g2, sem_out):
    cid = lax.axis_index("c")
    sid = lax.axis_index("s")
    wid = sid * NC + cid
    b0 = wid * SEQ_W

    pltpu.sync_copy(seq_hbm.at[pl.ds(b0, SEQ_W)], tok_idx_v)
    pltpu.sync_copy(seg_hbm.at[pl.ds(b0, SEQ_W)], seg_raw_v)

    lane = lax.iota(jnp.int32, LANES)
    col0 = [j * LANES for j in range(S // LANES)] + [S - LANES]

    def idx_body(r, _):
        for j0 in col0:
            pos3 = (lane + j0) * 3
            cmb_idx_v[r, pl.ds(j0, LANES)] = pos3 + seg_raw_v[r, pl.ds(j0, LANES)]
        return 0

    lax.fori_loop(0, SEQ_W, idx_body, 0)

    def fire_g1(c, p):
        s0 = c * SEQ_CH
        for s in range(SEQ_CH):
            for off, w in IDX_SPLIT:
                pltpu.async_copy(
                    comb_hbm.at[cmb_idx_v.at[s0 + s, pl.ds(off, w)]],
                    rows_v.at[p, s, pl.ds(off, w)], sem_g1)

    def drain(sem):
        pltpu.make_async_copy(out_hbm.at[pl.ds(0, SEQ_CH)], rows_v.at[0], sem).wait()

    fire_g1(0, 0)

    def chunk_body(c, _):
        p = lax.rem(c, 2)
        s0 = c * SEQ_CH
        drain(sem_g1)
        for s in range(SEQ_CH):
            for off, w in IDX_SPLIT:
                pltpu.async_copy(
                    tok_hbm.at[tok_idx_v.at[s0 + s, pl.ds(off, w)]],
                    rows_v.at[p, s, pl.ds(off, w)], sem_g2, add=True)

        @pl.when(c >= 1)
        def _():
            drain(sem_out)

        @pl.when(c <= NCHUNK - 2)
        def _():
            fire_g1(c + 1, 1 - p)

        drain(sem_g2)
        pltpu.async_copy(rows_v.at[p], out_hbm.at[pl.ds(b0 + s0, SEQ_CH)], sem_out)
        return 0

    lax.fori_loop(0, NCHUNK, chunk_body, 0)
    drain(sem_out)


@jax.jit
def kernel(sequence, segment_label, token_table, segment_table, pe):
    comb = (pe[:, None, :] + segment_table[None, :, :]).reshape(S * 3, E)
    tok_c = _jl.with_layout_constraint(
        token_table, _jl.Layout(major_to_minor=(0, 1), tiling=((8,),)))
    out = _sc_embed(sequence.astype(jnp.int32), segment_label.astype(jnp.int32),
                    tok_c, comb)
    return _jl.with_layout_constraint(
        out, _jl.Layout(major_to_minor=(0, 1, 2), tiling=((8,),)))


# speedup vs baseline: 1.3807x; 1.0001x over previous
import functools

import jax
import jax.numpy as jnp
from jax import lax
from jax.experimental import pallas as pl
from jax.experimental.pallas import tpu as pltpu
from jax.experimental.pallas import tpu_sc as plsc
from jax.experimental import layout as _jl

NC, NS, LANES = 2, 16, 16
NW = NC * NS
B, S, E = 1024, 200, 64
SEQ_W = B // NW
IDX_SPLIT = ((0, 104), (104, 96))
SEQ_CH = 2
NCHUNK = SEQ_W // SEQ_CH

_mesh = plsc.VectorSubcoreMesh(core_axis_name="c", subcore_axis_name="s")


@functools.partial(
    pl.kernel,
    out_type=jax.ShapeDtypeStruct((B, S, E), jnp.float32),
    mesh=_mesh,
    scratch_types=[
        pltpu.VMEM((SEQ_W, S), jnp.int32),
        pltpu.VMEM((SEQ_W, S), jnp.int32),
        pltpu.VMEM((SEQ_W, S), jnp.int32),
        pltpu.VMEM((2, SEQ_CH, S, E), jnp.float32),
        pltpu.SemaphoreType.DMA,
        pltpu.SemaphoreType.DMA,
        pltpu.SemaphoreType.DMA,
    ],
    compiler_params=pltpu.CompilerParams(use_tc_tiling_on_sc=False),
)
def _sc_embed(seq_hbm, seg_hbm, tok_hbm, comb_hbm, out_hbm,
              tok_idx_v, seg_raw_v, cmb_idx_v, rows_v, sem_g1, sem_g2, sem_out):
    cid = lax.axis_index("c")
    sid = lax.axis_index("s")
    wid = sid * NC + cid
    b0 = wid * SEQ_W

    pltpu.sync_copy(seq_hbm.at[pl.ds(b0, SEQ_W)], tok_idx_v)
    pltpu.sync_copy(seg_hbm.at[pl.ds(b0, SEQ_W)], seg_raw_v)

    lane = lax.iota(jnp.int32, LANES)
    col0 = [j * LANES for j in range(S // LANES)] + [S - LANES]

    def idx_body(r, _):
        for j0 in col0:
            pos3 = (lane + j0) * 3
            cmb_idx_v[r, pl.ds(j0, LANES)] = pos3 + seg_raw_v[r, pl.ds(j0, LANES)]
        return 0

    lax.fori_loop(0, SEQ_W, idx_body, 0)

    def fire_g1(c, p):
        s0 = c * SEQ_CH
        for s in range(SEQ_CH):
            for off, w in IDX_SPLIT:
                pltpu.async_copy(
                    comb_hbm.at[cmb_idx_v.at[s0 + s, pl.ds(off, w)]],
                    rows_v.at[p, s, pl.ds(off, w)], sem_g1)

    def drain(sem):
        pltpu.make_async_copy(out_hbm.at[pl.ds(0, SEQ_CH)], rows_v.at[0], sem).wait()

    fire_g1(0, 0)

    def chunk_body(c, _):
        p = lax.rem(c, 2)
        s0 = c * SEQ_CH
        drain(sem_g1)
        for s in range(SEQ_CH):
            for off, w in IDX_SPLIT:
                pltpu.async_copy(
                    tok_hbm.at[tok_idx_v.at[s0 + s, pl.ds(off, w)]],
                    rows_v.at[p, s, pl.ds(off, w)], sem_g2, add=True)

        @pl.when(c >= 1)
        def _():
            drain(sem_out)

        @pl.when(c <= NCHUNK - 2)
        def _():
            fire_g1(c + 1, 1 - p)

        drain(sem_g2)
        pltpu.async_copy(rows_v.at[p], out_hbm.at[pl.ds(b0 + s0, SEQ_CH)], sem_out)
        return 0

    lax.fori_loop(0, NCHUNK, chunk_body, 0)
    drain(sem_out)


@jax.jit
def kernel(sequence, segment_label, token_table, segment_table, pe):
    comb = (pe[:, None, :] + segment_table[None, :, :]).reshape(S * 3, E)
    tok_c = _jl.with_layout_constraint(
        token_table, _jl.Layout(major_to_minor=(0, 1), tiling=()))
    out = _sc_embed(sequence.astype(jnp.int32), segment_label.astype(jnp.int32),
                    tok_c, comb)
    return _jl.with_layout_constraint(
        out, _jl.Layout(major_to_minor=(0, 1, 2), tiling=((8,),)))
